# baseline (speedup 1.0000x reference)
import jax
import jax.numpy as jnp
import numpy as np
from jax.experimental import pallas as pl
from jax.experimental.pallas import tpu as pltpu

_B = 8192
_BB = 128
_F32 = jnp.float32


def _dot(a, b, out_dtype=_F32):
    return jax.lax.dot_general(a, b, (((1,), (0,)), ((), ())),
                               preferred_element_type=out_dtype)


def _dot00(a, b):
    return jax.lax.dot_general(a, b, (((0,), (0,)), ((), ())),
                               preferred_element_type=_F32)


def _row_body(db1_ref, dw2_ref, db2_ref, row_ref):
    row_ref[...] = _dot(jnp.maximum(db1_ref[...], 0.0),
                        dw2_ref[...]) + db2_ref[...]


def _pool_lane_pairs(x, bb):
    npair = x.shape[1] // (2 * bb)
    even = jnp.concatenate(
        [x[:, (2 * k) * bb:(2 * k + 1) * bb] for k in range(npair)], axis=1)
    odd = jnp.concatenate(
        [x[:, (2 * k + 1) * bb:(2 * k + 2) * bb] for k in range(npair)],
        axis=1)
    return jnp.maximum(even, odd)


def _shift_w(x, delta, bb):
    n = x.shape[1]
    if delta == 0:
        return x
    z = jnp.zeros((x.shape[0], abs(delta) * bb), x.dtype)
    if delta > 0:
        return jnp.concatenate([x[:, delta * bb:], z], axis=1)
    return jnp.concatenate([z, x[:, :n + delta * bb]], axis=1)


def _main_body(x_ref, cf_ref, w1_ref, b1_ref, w2_ref, b2_ref,
               cw1_ref, cb1_ref, cw2_ref, cb2_ref, row_ref,
               out_ref, fused_ref):
    bb = _BB
    n1 = 32 * bb
    n2 = 16 * bb
    x = x_ref[...].reshape(32, n1)
    xs = jnp.concatenate([_shift_w(x, d - 1, bb) for d in range(3)], axis=0)
    nq = n1 // 2
    quarters = []
    for qi in range(2):
        y1 = _dot(w1_ref[...], xs[:, qi * nq:(qi + 1) * nq])
        t = jnp.max(y1.reshape(16, 2, 32, nq), axis=1).reshape(512, nq)
        quarters.append(_pool_lane_pairs(t, bb))
    t = jnp.concatenate(quarters, axis=1)
    t = jnp.maximum(t + b1_ref[...], 0.0).astype(jnp.bfloat16)
    zr = jnp.zeros((32, n2), jnp.bfloat16)
    hp = jnp.concatenate([zr, t, zr], axis=0)
    sh = [_shift_w(hp, d - 1, bb) for d in range(3)]
    w2all = w2_ref[...]
    b2c = b2_ref[...]
    enc = jnp.zeros((64, bb), _F32)
    for g in range(8):
        acc = _dot00(w2all[0], sh[0][64 * g:64 * g + 128])
        acc += _dot00(w2all[1], sh[1][64 * g:64 * g + 128])
        acc += _dot00(w2all[2], sh[2][64 * g:64 * g + 128])
        p = jnp.maximum(acc[:64, :], acc[64:, :])
        p = _pool_lane_pairs(p, bb)
        p = jnp.maximum(p + b2c, 0.0)
        for k in range(8):
            enc = enc + p[:, k * bb:(k + 1) * bb]
    enc = enc * (1.0 / 64.0)
    cf = cf_ref[...].T
    c = jnp.maximum(_dot(cw1_ref[...], cf) + cb1_ref[...], 0.0)
    c = _dot(cw2_ref[...], c) + cb2_ref[...]
    fused_ref[...] = (enc + c).T
    out_ref[...] = jnp.broadcast_to(row_ref[...], (bb, 4096))


def _build_w1(conv1_w):
    bm = jnp.zeros((32, 32, 3, 32), _F32)
    h = np.arange(32)
    for di in range(3):
        hs = h[(h + di - 1 >= 0) & (h + di - 1 <= 31)]
        val = conv1_w[:, 0, di, :]
        bm = bm.at[hs, :, :, hs + di - 1].set(val[None])
    return bm.reshape(1024, 96)


def _build_w2(conv2_w):
    wm = jnp.zeros((3, 4, 32, 2, 64), _F32)
    for q in range(2):
        for hh in range(4):
            di = hh - q
            if 0 <= di <= 2:
                wm = wm.at[:, hh, :, q, :].set(
                    conv2_w[:, :, di, :].transpose(2, 1, 0))
    return wm.reshape(3, 128, 128)


def kernel(state, conceptual_features, conv1_w, conv1_b, conv2_w, conv2_b,
           ce_w1, ce_b1, ce_w2, ce_b2, router_w, dec_w1, dec_b1, dec_w2,
           dec_b2):
    del router_w, dec_w1
    nblk = _B // _BB
    xt = state.reshape(nblk, _BB, 32, 32).transpose(0, 2, 3, 1)
    xt = xt.reshape(nblk, 32, 32 * _BB).astype(jnp.bfloat16)
    w1 = _build_w1(conv1_w)
    b1c = jnp.tile(conv1_b, 16).reshape(512, 1)
    w2 = _build_w2(conv2_w)

    row = pl.pallas_call(
        _row_body,
        out_shape=jax.ShapeDtypeStruct((1, 4096), _F32),
    )(dec_b1.reshape(1, 256), dec_w2, dec_b2.reshape(1, 4096))

    zero2 = lambda i: (0, 0)
    out, fused = pl.pallas_call(
        _main_body,
        grid=(nblk,),
        in_specs=[
            pl.BlockSpec((1, 32, 32 * _BB), lambda i: (i, 0, 0)),
            pl.BlockSpec((_BB, 64), lambda i: (i, 0)),
            pl.BlockSpec((1024, 96), zero2),
            pl.BlockSpec((512, 1), zero2),
            pl.BlockSpec((3, 128, 128), lambda i: (0, 0, 0)),
            pl.BlockSpec((64, 1), zero2),
            pl.BlockSpec((64, 64), zero2),
            pl.BlockSpec((64, 1), zero2),
            pl.BlockSpec((64, 64), zero2),
            pl.BlockSpec((64, 1), zero2),
            pl.BlockSpec((1, 4096), zero2),
        ],
        out_specs=[
            pl.BlockSpec((_BB, 4096), lambda i: (i, 0)),
            pl.BlockSpec((_BB, 64), lambda i: (i, 0)),
        ],
        out_shape=[
            jax.ShapeDtypeStruct((_B, 4096), _F32),
            jax.ShapeDtypeStruct((_B, 64), _F32),
        ],
        compiler_params=pltpu.CompilerParams(
            dimension_semantics=("arbitrary",)),
    )(xt, conceptual_features, w1.astype(jnp.bfloat16), b1c,
      w2.astype(jnp.bfloat16), conv2_b.reshape(64, 1),
      ce_w1.T, ce_b1.reshape(64, 1), ce_w2.T, ce_b2.reshape(64, 1), row)

    return (out, fused, jnp.zeros((), _F32))


# speedup vs baseline: 1.0143x; 1.0143x over previous
import jax
import jax.numpy as jnp
import numpy as np
from jax.experimental import pallas as pl
from jax.experimental.pallas import tpu as pltpu

_B = 8192
_BB = 128
_F32 = jnp.float32


def _dot(a, b, out_dtype=_F32):
    return jax.lax.dot_general(a, b, (((1,), (0,)), ((), ())),
                               preferred_element_type=out_dtype)


def _dot00(a, b):
    return jax.lax.dot_general(a, b, (((0,), (0,)), ((), ())),
                               preferred_element_type=_F32)


def _row_body(db1_ref, dw2_ref, db2_ref, row_ref):
    row_ref[...] = _dot(jnp.maximum(db1_ref[...], 0.0),
                        dw2_ref[...]) + db2_ref[...]


def _pool_lane_pairs(x, bb):
    npair = x.shape[1] // (2 * bb)
    even = jnp.concatenate(
        [x[:, (2 * k) * bb:(2 * k + 1) * bb] for k in range(npair)], axis=1)
    odd = jnp.concatenate(
        [x[:, (2 * k + 1) * bb:(2 * k + 2) * bb] for k in range(npair)],
        axis=1)
    return jnp.maximum(even, odd)


def _shift_w(x, delta, bb):
    n = x.shape[1]
    if delta == 0:
        return x
    z = jnp.zeros((x.shape[0], abs(delta) * bb), x.dtype)
    if delta > 0:
        return jnp.concatenate([x[:, delta * bb:], z], axis=1)
    return jnp.concatenate([z, x[:, :n + delta * bb]], axis=1)


def _main_body(x_ref, cf_ref, w1_ref, b1_ref, w2_ref, b2_ref,
               cw1_ref, cb1_ref, cw2_ref, cb2_ref, row_ref,
               out_ref, fused_ref):
    bb = _BB
    n1 = 32 * bb
    n2 = 16 * bb
    x = x_ref[...].reshape(32, n1)
    xs = jnp.concatenate([_shift_w(x, d - 1, bb) for d in range(3)], axis=0)
    nq = n1 // 4
    quarters = []
    for qi in range(4):
        y1 = _dot(w1_ref[...], xs[:, qi * nq:(qi + 1) * nq])
        t = jnp.max(y1.reshape(16, 2, 32, nq), axis=1).reshape(512, nq)
        quarters.append(_pool_lane_pairs(t, bb))
    t = jnp.concatenate(quarters, axis=1)
    t = jnp.maximum(t + b1_ref[...], 0.0).astype(jnp.bfloat16)
    zr = jnp.zeros((32, n2), jnp.bfloat16)
    hp = jnp.concatenate([zr, t, zr], axis=0)
    sh = [_shift_w(hp, d - 1, bb) for d in range(3)]
    w2all = w2_ref[...]
    b2c = b2_ref[...]
    enc = jnp.zeros((64, bb), _F32)
    for g in range(8):
        acc = _dot00(w2all[0], sh[0][64 * g:64 * g + 128])
        acc += _dot00(w2all[1], sh[1][64 * g:64 * g + 128])
        acc += _dot00(w2all[2], sh[2][64 * g:64 * g + 128])
        p = jnp.maximum(acc[:64, :], acc[64:, :])
        p = _pool_lane_pairs(p, bb)
        p = jnp.maximum(p + b2c, 0.0)
        for k in range(8):
            enc = enc + p[:, k * bb:(k + 1) * bb]
    enc = enc * (1.0 / 64.0)
    cf = cf_ref[...].reshape(64, bb)
    c = jnp.maximum(_dot(cw1_ref[...], cf) + cb1_ref[...], 0.0)
    c = _dot(cw2_ref[...], c) + cb2_ref[...]
    fused_ref[...] = (enc + c).reshape(1, 64, bb)
    out_ref[...] = jnp.broadcast_to(row_ref[...], (bb, 4096))


def _build_w1(conv1_w):
    h = np.arange(32)
    d = h[None, :] - h[:, None] + 1
    mask = jnp.asarray((d >= 0) & (d <= 2), _F32)
    wt = conv1_w[:, 0].transpose(1, 0, 2)
    g = wt[np.clip(d, 0, 2)]
    bm = g.transpose(0, 2, 3, 1) * mask[:, None, None, :]
    return bm.reshape(1024, 96)


def _build_w2(conv2_w):
    hh = np.arange(4)
    d2 = hh[:, None] - np.arange(2)[None, :]
    mask = jnp.asarray((d2 >= 0) & (d2 <= 2), _F32)
    wc = conv2_w[:, :, np.clip(d2, 0, 2), :]
    wm = wc.transpose(4, 2, 1, 3, 0) * mask[None, :, None, :, None]
    return wm.reshape(3, 128, 128)


def kernel(state, conceptual_features, conv1_w, conv1_b, conv2_w, conv2_b,
           ce_w1, ce_b1, ce_w2, ce_b2, router_w, dec_w1, dec_b1, dec_w2,
           dec_b2):
    del router_w, dec_w1
    nblk = _B // _BB
    xt = state.reshape(nblk, _BB, 32, 32).transpose(0, 2, 3, 1)
    xt = xt.reshape(nblk, 32, 32 * _BB).astype(jnp.bfloat16)
    cft = conceptual_features.reshape(nblk, _BB, 64).transpose(0, 2, 1)
    w1 = _build_w1(conv1_w)
    b1c = jnp.tile(conv1_b, 16).reshape(512, 1)
    w2 = _build_w2(conv2_w)

    row = pl.pallas_call(
        _row_body,
        out_shape=jax.ShapeDtypeStruct((1, 4096), _F32),
    )(dec_b1.reshape(1, 256), dec_w2, dec_b2.reshape(1, 4096))

    zero2 = lambda i: (0, 0)
    out, fused_t = pl.pallas_call(
        _main_body,
        grid=(nblk,),
        in_specs=[
            pl.BlockSpec((1, 32, 32 * _BB), lambda i: (i, 0, 0)),
            pl.BlockSpec((1, 64, _BB), lambda i: (i, 0, 0)),
            pl.BlockSpec((1024, 96), zero2),
            pl.BlockSpec((512, 1), zero2),
            pl.BlockSpec((3, 128, 128), lambda i: (0, 0, 0)),
            pl.BlockSpec((64, 1), zero2),
            pl.BlockSpec((64, 64), zero2),
            pl.BlockSpec((64, 1), zero2),
            pl.BlockSpec((64, 64), zero2),
            pl.BlockSpec((64, 1), zero2),
            pl.BlockSpec((1, 4096), zero2),
        ],
        out_specs=[
            pl.BlockSpec((_BB, 4096), lambda i: (i, 0)),
            pl.BlockSpec((1, 64, _BB), lambda i: (i, 0, 0)),
        ],
        out_shape=[
            jax.ShapeDtypeStruct((_B, 4096), _F32),
            jax.ShapeDtypeStruct((nblk, 64, _BB), _F32),
        ],
        compiler_params=pltpu.CompilerParams(
            dimension_semantics=("arbitrary",)),
    )(xt, cft, w1.astype(jnp.bfloat16), b1c,
      w2.astype(jnp.bfloat16), conv2_b.reshape(64, 1),
      ce_w1.T, ce_b1.reshape(64, 1), ce_w2.T, ce_b2.reshape(64, 1), row)

    fused = fused_t.transpose(0, 2, 1).reshape(_B, 64)
    return (out, fused, jnp.zeros((), _F32))


# speedup vs baseline: 1.0215x; 1.0070x over previous
import jax
import jax.numpy as jnp
import numpy as np
from jax.experimental import pallas as pl
from jax.experimental.pallas import tpu as pltpu

_B = 8192
_BB = 128
_F32 = jnp.float32


def _dot(a, b, out_dtype=_F32):
    return jax.lax.dot_general(a, b, (((1,), (0,)), ((), ())),
                               preferred_element_type=out_dtype)


def _dot00(a, b):
    return jax.lax.dot_general(a, b, (((0,), (0,)), ((), ())),
                               preferred_element_type=_F32)


def _row_body(db1_ref, dw2_ref, db2_ref, row_ref):
    row_ref[...] = _dot(jnp.maximum(db1_ref[...], 0.0),
                        dw2_ref[...]) + db2_ref[...]


def _pool_lane_pairs(x, bb):
    npair = x.shape[1] // (2 * bb)
    even = jnp.concatenate(
        [x[:, (2 * k) * bb:(2 * k + 1) * bb] for k in range(npair)], axis=1)
    odd = jnp.concatenate(
        [x[:, (2 * k + 1) * bb:(2 * k + 2) * bb] for k in range(npair)],
        axis=1)
    return jnp.maximum(even, odd)


def _shift_w(x, delta, bb):
    n = x.shape[1]
    if delta == 0:
        return x
    z = jnp.zeros((x.shape[0], abs(delta) * bb), x.dtype)
    if delta > 0:
        return jnp.concatenate([x[:, delta * bb:], z], axis=1)
    return jnp.concatenate([z, x[:, :n + delta * bb]], axis=1)


def _main_body(x_ref, cf_ref, w1_ref, b1_ref, w2_ref, b2_ref,
               cw1_ref, cb1_ref, cw2_ref, cb2_ref, row_ref,
               out_ref, fused_ref):
    bb = _BB
    n1 = 32 * bb
    n2 = 16 * bb
    x = x_ref[...].reshape(32, n1)
    xs = jnp.concatenate([_shift_w(x, d - 1, bb) for d in range(3)], axis=0)
    nq = n1 // 4
    quarters = []
    for qi in range(4):
        y1 = _dot(w1_ref[...], xs[:, qi * nq:(qi + 1) * nq])
        t = jnp.max(y1.reshape(16, 2, 32, nq), axis=1).reshape(512, nq)
        quarters.append(_pool_lane_pairs(t, bb))
    t = jnp.concatenate(quarters, axis=1)
    t = jnp.maximum(t + b1_ref[...], 0.0).astype(jnp.bfloat16)
    zr = jnp.zeros((32, n2), jnp.bfloat16)
    hp = jnp.concatenate([zr, t, zr], axis=0)
    sh = [_shift_w(hp, d - 1, bb) for d in range(3)]
    w2all = w2_ref[...]
    b2c = b2_ref[...]
    enc = jnp.zeros((64, bb), _F32)
    for g in range(8):
        acc = _dot00(w2all[0], sh[0][64 * g:64 * g + 128])
        acc += _dot00(w2all[1], sh[1][64 * g:64 * g + 128])
        acc += _dot00(w2all[2], sh[2][64 * g:64 * g + 128])
        p = jnp.maximum(acc[:64, :], acc[64:, :])
        p = _pool_lane_pairs(p, bb)
        p = jnp.maximum(p + b2c, 0.0)
        for k in range(8):
            enc = enc + p[:, k * bb:(k + 1) * bb]
    enc = enc * (1.0 / 64.0)
    cf = cf_ref[...].reshape(64, bb)
    c = jnp.maximum(_dot(cw1_ref[...], cf) + cb1_ref[...], 0.0)
    c = _dot(cw2_ref[...], c) + cb2_ref[...]
    fused_ref[...] = (enc + c).reshape(1, 64, bb)
    out_ref[...] = jnp.broadcast_to(row_ref[...], (bb, 4096))


def _build_w1(conv1_w):
    bm = jnp.zeros((32, 32, 3, 32), _F32)
    h = np.arange(32)
    for di in range(3):
        hs = h[(h + di - 1 >= 0) & (h + di - 1 <= 31)]
        val = conv1_w[:, 0, di, :]
        bm = bm.at[hs, :, :, hs + di - 1].set(val[None])
    return bm.reshape(1024, 96)


def _build_w2(conv2_w):
    wm = jnp.zeros((3, 4, 32, 2, 64), _F32)
    for q in range(2):
        for hh in range(4):
            di = hh - q
            if 0 <= di <= 2:
                wm = wm.at[:, hh, :, q, :].set(
                    conv2_w[:, :, di, :].transpose(2, 1, 0))
    return wm.reshape(3, 128, 128)


def kernel(state, conceptual_features, conv1_w, conv1_b, conv2_w, conv2_b,
           ce_w1, ce_b1, ce_w2, ce_b2, router_w, dec_w1, dec_b1, dec_w2,
           dec_b2):
    del router_w, dec_w1
    nblk = _B // _BB
    xt = state.reshape(nblk, _BB, 32, 32).transpose(0, 2, 3, 1)
    xt = xt.reshape(nblk, 32, 32 * _BB).astype(jnp.bfloat16)
    cft = conceptual_features.reshape(nblk, _BB, 64).transpose(0, 2, 1)
    w1 = _build_w1(conv1_w)
    b1c = jnp.tile(conv1_b, 16).reshape(512, 1)
    w2 = _build_w2(conv2_w)

    row = pl.pallas_call(
        _row_body,
        out_shape=jax.ShapeDtypeStruct((1, 4096), _F32),
    )(dec_b1.reshape(1, 256), dec_w2, dec_b2.reshape(1, 4096))

    zero2 = lambda i: (0, 0)
    out, fused_t = pl.pallas_call(
        _main_body,
        grid=(nblk,),
        in_specs=[
            pl.BlockSpec((1, 32, 32 * _BB), lambda i: (i, 0, 0)),
            pl.BlockSpec((1, 64, _BB), lambda i: (i, 0, 0)),
            pl.BlockSpec((1024, 96), zero2),
            pl.BlockSpec((512, 1), zero2),
            pl.BlockSpec((3, 128, 128), lambda i: (0, 0, 0)),
            pl.BlockSpec((64, 1), zero2),
            pl.BlockSpec((64, 64), zero2),
            pl.BlockSpec((64, 1), zero2),
            pl.BlockSpec((64, 64), zero2),
            pl.BlockSpec((64, 1), zero2),
            pl.BlockSpec((1, 4096), zero2),
        ],
        out_specs=[
            pl.BlockSpec((_BB, 4096), lambda i: (i, 0)),
            pl.BlockSpec((1, 64, _BB), lambda i: (i, 0, 0)),
        ],
        out_shape=[
            jax.ShapeDtypeStruct((_B, 4096), _F32),
            jax.ShapeDtypeStruct((nblk, 64, _BB), _F32),
        ],
        compiler_params=pltpu.CompilerParams(
            dimension_semantics=("arbitrary",)),
    )(xt, cft, w1.astype(jnp.bfloat16), b1c,
      w2.astype(jnp.bfloat16), conv2_b.reshape(64, 1),
      ce_w1.T, ce_b1.reshape(64, 1), ce_w2.T, ce_b2.reshape(64, 1), row)

    fused = fused_t.transpose(0, 2, 1).reshape(_B, 64)
    return (out, fused, jnp.zeros((), _F32))
